# baseline (speedup 1.0000x reference)
import functools

import jax
import jax.numpy as jnp
from jax import lax
from jax.experimental import pallas as pl
from jax.experimental.pallas import tpu as pltpu
from jax.experimental.pallas import tpu_sc as plsc

_ORG_VOCAB = 100000
_EXTRA_VOCAB = 256
_EMBED_DIM = 2048
_MAX_LORAS = 8
_RANK = 16

_NC, _NS = 2, 16
_NW = _NC * _NS
_CHUNK = 8
_AGRP = 8


def _sc_gather_build(n_tok: int):
    tpw = n_tok // _NW
    nch = tpw // _CHUNK
    ngr = tpw // _AGRP
    assert ngr == nch
    mesh = plsc.VectorSubcoreMesh(core_axis_name="c", subcore_axis_name="s")

    @functools.partial(
        pl.kernel,
        out_type=[
            jax.ShapeDtypeStruct((n_tok, _EMBED_DIM), jnp.float32),
            jax.ShapeDtypeStruct((_RANK, n_tok), jnp.float32),
        ],
        mesh=mesh,
        compiler_params=pltpu.CompilerParams(needs_layout_passes=False),
        scratch_types=[
            pltpu.VMEM((tpw,), jnp.int32),
            pltpu.VMEM((tpw,), jnp.int32),
            pltpu.VMEM((_RANK, tpw), jnp.float32),
            pltpu.VMEM((_CHUNK, _EMBED_DIM), jnp.float32),
            pltpu.VMEM((_CHUNK, _EMBED_DIM), jnp.float32),
            pltpu.VMEM((_AGRP, _RANK, 128), jnp.float32),
            pltpu.VMEM((_AGRP, _RANK, 128), jnp.float32),
            pltpu.SemaphoreType.DMA,
            pltpu.SemaphoreType.DMA,
            pltpu.SemaphoreType.DMA,
            pltpu.SemaphoreType.DMA,
        ],
    )
    def sc_gather(base_hbm, at_hbm, idx_hbm, lidx_hbm, rows_out, at_out,
                  idx_v, lidx_v, a_vt, buf0, buf1, slab0, slab1,
                  sem0, sem1, sema0, sema1):
        wid = lax.axis_index("s") * _NC + lax.axis_index("c")
        base = wid * tpw
        pltpu.sync_copy(idx_hbm.at[pl.ds(base, tpw)], idx_v)
        pltpu.sync_copy(lidx_hbm.at[pl.ds(base, tpw)], lidx_v)
        row = lax.iota(jnp.int32, 16)
        slabs = (slab0, slab1)
        semas = (sema0, sema1)

        bufs = (buf0, buf1)
        sems = (sem0, sem1)

        def fire_base(c, k):
            pltpu.async_copy(
                base_hbm.at[idx_v.at[pl.ds(c * _CHUNK, _CHUNK)]],
                bufs[k], sems[k])

        def wait_base(k):
            pltpu.make_async_copy(
                base_hbm.at[idx_v.at[pl.ds(0, _CHUNK)]],
                bufs[k], sems[k]).wait()

        def fire_a(g, k):
            off = (g // 2) * 16
            l_vec = lidx_v[pl.ds(off, 16)]
            v_vec = idx_v[pl.ds(off, 16)]
            for j in range(_AGRP):
                jj = j + _AGRP * k
                v0 = pl.multiple_of((v_vec[jj] >> 7) << 7, 128)
                pltpu.async_copy(
                    at_hbm.at[l_vec[jj], :, pl.ds(v0, 128)],
                    slabs[k].at[j], semas[k])

        def wait_a(k):
            for j in range(_AGRP):
                pltpu.make_async_copy(
                    at_hbm.at[0, :, pl.ds(0, 128)],
                    slabs[k].at[j], semas[k]).wait()

        def extract_a(g, k):
            v_vec = idx_v[pl.ds((g // 2) * 16, 16)]
            for j in range(_AGRP):
                jj = j + _AGRP * k
                t = g * _AGRP + j
                col = jnp.full((16,), v_vec[jj] & 127, jnp.int32)
                val = plsc.load_gather(
                    slabs[k], [jnp.full((16,), j, jnp.int32), row, col])
                plsc.store_scatter(
                    a_vt, [row, jnp.full((16,), t, jnp.int32)], val)

        half = nch // 2
        fire_base(0, 0)
        fire_a(0, 0)

        def body(i, carry):
            c0 = 2 * i
            fire_base(c0 + 1, 1)
            fire_a(c0 + 1, 1)
            wait_base(0)
            pltpu.sync_copy(bufs[0],
                            rows_out.at[pl.ds(base + c0 * _CHUNK, _CHUNK)])
            wait_a(0)
            extract_a(c0, 0)

            @pl.when(i + 1 < half)
            def _():
                fire_base(c0 + 2, 0)
                fire_a(c0 + 2, 0)

            wait_base(1)
            pltpu.sync_copy(
                bufs[1],
                rows_out.at[pl.ds(base + (c0 + 1) * _CHUNK, _CHUNK)])
            wait_a(1)
            extract_a(c0 + 1, 1)
            return carry

        lax.fori_loop(0, half, body, 0)
        pltpu.sync_copy(a_vt, at_out.at[:, pl.ds(base, tpw)])

    return sc_gather


def _tc_compute(rows_ref, at_ref, idx_ref, bt_ref, out_ref):
    a = jnp.transpose(at_ref[...], (1, 0))
    idx = idx_ref[...]
    bt_blk = a.shape[0]
    cols = lax.broadcasted_iota(jnp.int32, (bt_blk, _MAX_LORAS * _RANK), 1)
    sel = (cols // _RANK) == idx
    a_exp = jnp.where(sel, jnp.concatenate([a] * _MAX_LORAS, axis=1), 0.0)
    delta = jnp.dot(a_exp.astype(jnp.bfloat16), bt_ref[...],
                    preferred_element_type=jnp.float32)
    out_ref[...] = rows_ref[...] + delta


def _tc_body(rows_ref, at_ref, idx_ref, bt_ref, out_ref):
    _tc_compute(rows_ref, at_ref, idx_ref, bt_ref, out_ref)


def _tc_body2(rows_ref, at_ref, idx_ref, bt_ref, full_ref, out_ref):
    del full_ref
    _tc_compute(rows_ref, at_ref, idx_ref, bt_ref, out_ref)


def kernel(x, base_weight, lora_a_stacked, lora_b_stacked, base_indices,
           embeddings_indices):
    b, s = x.shape
    n_tok = b * s
    xf = x.reshape(n_tok).astype(jnp.int32)
    lidx = (embeddings_indices[1][:n_tok]
            // (_ORG_VOCAB + _EXTRA_VOCAB)).astype(jnp.int32)

    at3 = jnp.transpose(lora_a_stacked, (0, 2, 1))
    bt2 = lora_b_stacked[:, 0].transpose(0, 2, 1).reshape(
        _MAX_LORAS * _RANK, _EMBED_DIM).astype(jnp.bfloat16)

    idx2 = base_indices[:n_tok].reshape(n_tok, 1).astype(jnp.int32)

    n_half = n_tok // 2
    sc_half = _sc_gather_build(n_half)
    rows0, at0 = sc_half(base_weight, at3, xf[:n_half], lidx[:n_half])
    rows1, at1 = sc_half(base_weight, at3, xf[n_half:], lidx[n_half:])

    bt_tok = 1024
    grid = (n_half // bt_tok,)
    nblk = n_half // bt_tok
    half_specs = [
        pl.BlockSpec((bt_tok, _EMBED_DIM), lambda i: (i, 0)),
        pl.BlockSpec((_RANK, bt_tok), lambda i: (0, i)),
        pl.BlockSpec((bt_tok, 1), lambda i: (i, 0)),
        pl.BlockSpec((_MAX_LORAS * _RANK, _EMBED_DIM), lambda i: (0, 0)),
    ]
    out_shape = jax.ShapeDtypeStruct((n_tok, _EMBED_DIM), jnp.float32)
    buf0 = pl.pallas_call(
        _tc_body,
        grid=grid,
        in_specs=half_specs,
        out_specs=pl.BlockSpec((bt_tok, _EMBED_DIM), lambda i: (i, 0)),
        out_shape=out_shape,
    )(rows0, at0, idx2[:n_half], bt2)
    out = pl.pallas_call(
        _tc_body2,
        grid=grid,
        in_specs=half_specs + [pl.BlockSpec(memory_space=pl.ANY)],
        out_specs=pl.BlockSpec((bt_tok, _EMBED_DIM),
                               lambda i, n=nblk: (i + n, 0)),
        out_shape=out_shape,
        input_output_aliases={4: 0},
    )(rows1, at1, idx2[n_half:], bt2, buf0)

    return out.reshape(b, s, _EMBED_DIM)


# speedup vs baseline: 1.0191x; 1.0191x over previous
import functools

import jax
import jax.numpy as jnp
from jax import lax
from jax.experimental import pallas as pl
from jax.experimental.pallas import tpu as pltpu
from jax.experimental.pallas import tpu_sc as plsc

_ORG_VOCAB = 100000
_EXTRA_VOCAB = 256
_EMBED_DIM = 2048
_MAX_LORAS = 8
_RANK = 16

_NC, _NS = 2, 16
_NW = _NC * _NS
_CHUNK = 8
_AGRP = 8


def _sc_gather_build(n_tok: int):
    tpw = n_tok // _NW
    nch = tpw // _CHUNK
    ngr = tpw // _AGRP
    assert ngr == nch
    mesh = plsc.VectorSubcoreMesh(core_axis_name="c", subcore_axis_name="s")

    @functools.partial(
        pl.kernel,
        out_type=[
            jax.ShapeDtypeStruct((n_tok, _EMBED_DIM), jnp.float32),
            jax.ShapeDtypeStruct((_RANK, n_tok), jnp.float32),
        ],
        mesh=mesh,
        compiler_params=pltpu.CompilerParams(needs_layout_passes=False),
        scratch_types=[
            pltpu.VMEM((tpw,), jnp.int32),
            pltpu.VMEM((tpw,), jnp.int32),
            pltpu.VMEM((_RANK, tpw), jnp.float32),
            pltpu.VMEM((_CHUNK, _EMBED_DIM), jnp.float32),
            pltpu.VMEM((_CHUNK, _EMBED_DIM), jnp.float32),
            pltpu.VMEM((_AGRP, _RANK, 128), jnp.float32),
            pltpu.VMEM((_AGRP, _RANK, 128), jnp.float32),
            pltpu.SemaphoreType.DMA,
            pltpu.SemaphoreType.DMA,
            pltpu.SemaphoreType.DMA,
            pltpu.SemaphoreType.DMA,
        ],
    )
    def sc_gather(base_hbm, at_hbm, idx_hbm, lidx_hbm, rows_out, at_out,
                  idx_v, lidx_v, a_vt, buf0, buf1, slab0, slab1,
                  sem0, sem1, sema0, sema1):
        wid = lax.axis_index("s") * _NC + lax.axis_index("c")
        base = wid * tpw
        pltpu.sync_copy(idx_hbm.at[pl.ds(base, tpw)], idx_v)
        pltpu.sync_copy(lidx_hbm.at[pl.ds(base, tpw)], lidx_v)
        row = lax.iota(jnp.int32, 16)
        slabs = (slab0, slab1)
        semas = (sema0, sema1)

        bufs = (buf0, buf1)
        sems = (sem0, sem1)

        def fire_base(c, k):
            pltpu.async_copy(
                base_hbm.at[idx_v.at[pl.ds(c * _CHUNK, _CHUNK)]],
                bufs[k], sems[k])

        def wait_base(k):
            pltpu.make_async_copy(
                base_hbm.at[idx_v.at[pl.ds(0, _CHUNK)]],
                bufs[k], sems[k]).wait()

        def fire_a(g, k):
            off = (g // 2) * 16
            l_vec = lidx_v[pl.ds(off, 16)]
            v_vec = idx_v[pl.ds(off, 16)]
            for j in range(_AGRP):
                jj = j + _AGRP * k
                v0 = pl.multiple_of((v_vec[jj] >> 7) << 7, 128)
                pltpu.async_copy(
                    at_hbm.at[l_vec[jj], :, pl.ds(v0, 128)],
                    slabs[k].at[j], semas[k])

        def wait_a(k):
            for j in range(_AGRP):
                pltpu.make_async_copy(
                    at_hbm.at[0, :, pl.ds(0, 128)],
                    slabs[k].at[j], semas[k]).wait()

        def extract_a(g, k):
            v_vec = idx_v[pl.ds((g // 2) * 16, 16)]
            for j in range(_AGRP):
                jj = j + _AGRP * k
                t = g * _AGRP + j
                col = jnp.full((16,), v_vec[jj] & 127, jnp.int32)
                val = plsc.load_gather(
                    slabs[k], [jnp.full((16,), j, jnp.int32), row, col])
                plsc.store_scatter(
                    a_vt, [row, jnp.full((16,), t, jnp.int32)], val)

        half = nch // 2
        fire_base(0, 0)
        fire_a(0, 0)

        def body(i, carry):
            c0 = 2 * i
            fire_base(c0 + 1, 1)
            fire_a(c0 + 1, 1)
            wait_base(0)
            pltpu.sync_copy(bufs[0],
                            rows_out.at[pl.ds(base + c0 * _CHUNK, _CHUNK)])
            wait_a(0)
            extract_a(c0, 0)

            @pl.when(i + 1 < half)
            def _():
                fire_base(c0 + 2, 0)
                fire_a(c0 + 2, 0)

            wait_base(1)
            pltpu.sync_copy(
                bufs[1],
                rows_out.at[pl.ds(base + (c0 + 1) * _CHUNK, _CHUNK)])
            wait_a(1)
            extract_a(c0 + 1, 1)
            return carry

        lax.fori_loop(0, half, body, 0)
        pltpu.sync_copy(a_vt, at_out.at[:, pl.ds(base, tpw)])

    return sc_gather


def _tc_body(rows_ref, at_ref, idx_ref, bt_ref, out_ref):
    a = jnp.transpose(at_ref[...], (1, 0))
    idx = idx_ref[...]
    bt_blk = a.shape[0]
    cols = lax.broadcasted_iota(jnp.int32, (bt_blk, _MAX_LORAS * _RANK), 1)
    sel = (cols // _RANK) == idx
    a_exp = jnp.where(sel, jnp.concatenate([a] * _MAX_LORAS, axis=1), 0.0)
    delta = jnp.dot(a_exp.astype(jnp.bfloat16), bt_ref[...],
                    preferred_element_type=jnp.float32)
    out_ref[...] = rows_ref[...] + delta


def kernel(x, base_weight, lora_a_stacked, lora_b_stacked, base_indices,
           embeddings_indices):
    b, s = x.shape
    n_tok = b * s
    xf = x.reshape(n_tok).astype(jnp.int32)
    lidx = (embeddings_indices[1][:n_tok]
            // (_ORG_VOCAB + _EXTRA_VOCAB)).astype(jnp.int32)

    at3 = jnp.transpose(lora_a_stacked, (0, 2, 1))
    bt2 = lora_b_stacked[:, 0].transpose(0, 2, 1).reshape(
        _MAX_LORAS * _RANK, _EMBED_DIM).astype(jnp.bfloat16)

    idx2 = base_indices[:n_tok].reshape(n_tok, 1).astype(jnp.int32)

    rows, a_t = _sc_gather_build(n_tok)(base_weight, at3, xf, lidx)

    bt_tok = 1024
    out = pl.pallas_call(
        _tc_body,
        grid=(n_tok // bt_tok,),
        in_specs=[
            pl.BlockSpec((bt_tok, _EMBED_DIM), lambda i: (i, 0)),
            pl.BlockSpec((_RANK, bt_tok), lambda i: (0, i)),
            pl.BlockSpec((bt_tok, 1), lambda i: (i, 0)),
            pl.BlockSpec((_MAX_LORAS * _RANK, _EMBED_DIM), lambda i: (0, 0)),
        ],
        out_specs=pl.BlockSpec((bt_tok, _EMBED_DIM), lambda i: (i, 0)),
        out_shape=jax.ShapeDtypeStruct((n_tok, _EMBED_DIM), jnp.float32),
    )(rows, a_t, idx2, bt2)

    return out.reshape(b, s, _EMBED_DIM)
